# baseline (speedup 1.0000x reference)
import jax
import jax.numpy as jnp
from jax import lax
from jax.experimental import pallas as pl
from jax.experimental.pallas import tpu as pltpu

_TOPK = 80
_RADIUS_GAIN = 14.0
_RADIUS_MIN = 1.5
_RADIUS_MAX = 18.0
_SLAB = 120
_CSLAB = 256


def _shift(x, d, axis, fill):
    h, w = x.shape
    if d == 0:
        return x
    if axis == 1:
        pad = jnp.full((h, abs(d)), fill, x.dtype)
        if d > 0:
            return jnp.concatenate([pad, x[:, : w - d]], axis=1)
        return jnp.concatenate([x[:, -d:], pad], axis=1)
    pad = jnp.full((abs(d), w), fill, x.dtype)
    if d > 0:
        return jnp.concatenate([pad, x[: h - d, :]], axis=0)
    return jnp.concatenate([x[-d:, :], pad], axis=0)


def _sum9(x, axis):
    acc = x
    for d in (-4, -3, -2, -1, 1, 2, 3, 4):
        acc = acc + _shift(x, d, axis, 0.0)
    return acc


def _max5(x, axis):
    ninf = -jnp.inf
    m3 = jnp.maximum(x, jnp.maximum(_shift(x, 1, axis, ninf),
                                    _shift(x, -1, axis, ninf)))
    return jnp.maximum(_shift(m3, 1, axis, ninf), _shift(m3, -1, axis, ninf))


def _body(vs_ref, dp_ref, *refs):
    b = vs_ref.shape[0]
    out_refs = refs[:b]
    mx_refs = refs[b:]
    h, w = mx_refs[0].shape
    nb = h // 8

    ciota = lax.broadcasted_iota(jnp.int32, (1, w), 1)
    xxc = lax.broadcasted_iota(jnp.int32, (1, _CSLAB), 1).astype(jnp.float32)
    fiota = (lax.broadcasted_iota(jnp.int32, (8, nb), 0) * nb
             + lax.broadcasted_iota(jnp.int32, (8, nb), 1))
    yyf = lax.broadcasted_iota(jnp.int32, (_SLAB, 1), 0).astype(jnp.float32)
    wiota3 = lax.broadcasted_iota(jnp.int32, (8, nb, w), 2)

    rm0 = []
    rmc0 = []
    for bi in range(b):
        p = vs_ref[bi]
        occ = (p > 0.2).astype(jnp.float32)
        so = _sum9(_sum9(occ, 1), 0)
        sm = _sum9(_sum9(p, 1), 0)
        prod = so * sm * (1.0 / (81.0 * 81.0))
        mn = jnp.min(prod)
        mxv = jnp.max(prod)
        compact = (prod - mn) / (mxv - mn + 1e-6)
        score = p * compact
        pooled = _max5(_max5(score, 1), 0)
        maxima = jnp.where(score == pooled, score, 0.0)
        mx_refs[bi][:, :] = maxima
        m3 = maxima.reshape(8, nb, w)
        mr = jnp.max(m3, axis=2)
        rm0.append(mr)
        rmc0.append(jnp.min(jnp.where(m3 == mr[:, :, None], wiota3, w),
                            axis=2))
        out_refs[bi][:, :] = jnp.full((h, w), -jnp.inf, jnp.float32)

    def splat(bi, lv, rfl, ic, cfl, base, cbase):
        base = pl.multiple_of(base, 8)
        cbase = pl.multiple_of(cbase, 128)
        ay = lv - (yyf - rfl) ** 2 * ic
        bx = (xxc - cfl) ** 2 * ic
        cur = out_refs[bi][pl.ds(base, _SLAB), pl.ds(cbase, _CSLAB)]
        out_refs[bi][pl.ds(base, _SLAB), pl.ds(cbase, _CSLAB)] = (
            jnp.maximum(cur, ay - bx))

    def step1(bi, st):
        rm, rmc, plv, prfl, pic, pcfl, pbase, pcbase = st
        m = jnp.max(rm)
        r = jnp.min(jnp.where(rm == m, fiota, h))
        c = jnp.min(jnp.where(fiota == r, rmc, w))
        row = mx_refs[bi][pl.ds(r, 1), :]
        newrow = jnp.where(ciota == c, 0.0, row)
        mx_refs[bi][pl.ds(r, 1), :] = newrow
        nm = jnp.max(newrow)
        nc = jnp.min(jnp.where(newrow == nm, ciota, w))
        rm = jnp.where(fiota == r, nm, rm)
        rmc = jnp.where(fiota == r, nc, rmc)
        splat(bi, plv, prfl, pic, pcfl, pbase, pcbase)
        drow = dp_ref[bi, pl.ds(r, 1), :]
        z = jnp.sum(jnp.where(ciota == c, drow, 0.0))
        z = jnp.maximum(z, 0.001)
        radius = jnp.clip(_RADIUS_GAIN / z, _RADIUS_MIN, _RADIUS_MAX)
        sig2 = (0.6 * radius) ** 2
        invc = 1.0 / (2.0 * sig2 + 1e-6)
        base = pl.multiple_of((jnp.clip(r - _SLAB // 2, 0, h - _SLAB) // 8) * 8,
                              8)
        cbase = (jnp.clip(c - 56, 0, w - _CSLAB) // 128) * 128
        return (rm, rmc, jnp.log(m), (r - base).astype(jnp.float32), invc,
                (c - cbase).astype(jnp.float32), base, cbase)

    def step(i, sts):
        return tuple(step1(bi, sts[bi]) for bi in range(b))

    init = tuple((rm0[bi], rmc0[bi], -jnp.inf, 0.0, 0.0, 0.0, 0, 0)
                 for bi in range(b))
    fin = lax.fori_loop(0, _TOPK, step, init, unroll=2)

    for bi in range(b):
        splat(bi, *fin[bi][2:])
        g = jnp.exp(out_refs[bi][:, :])
        gmn = jnp.min(g)
        gmx = jnp.max(g)
        out_refs[bi][:, :] = (g - gmn) / (gmx - gmn + 1e-6)


def kernel(voxel_score_map, depth):
    b, ch, h, w = voxel_score_map.shape
    vs = voxel_score_map.reshape(b, h, w)
    dp = depth.reshape(b, h, w)
    outs = pl.pallas_call(
        _body,
        out_shape=[jax.ShapeDtypeStruct((h, w), jnp.float32)
                   for _ in range(b)],
        scratch_shapes=[pltpu.VMEM((h, w), jnp.float32) for _ in range(b)],
    )(vs, dp)
    return jnp.stack(outs, axis=0).reshape(b, ch, h, w)


# speedup vs baseline: 1.2846x; 1.2846x over previous
import jax
import jax.numpy as jnp
from jax import lax
from jax.experimental import pallas as pl
from jax.experimental.pallas import tpu as pltpu

_TOPK = 80
_RADIUS_GAIN = 14.0
_RADIUS_MIN = 1.5
_RADIUS_MAX = 18.0
_SLAB = 120
_CSLAB = 256


def _shift(x, d, axis, fill):
    h, w = x.shape
    if d == 0:
        return x
    if axis == 1:
        pad = jnp.full((h, abs(d)), fill, x.dtype)
        if d > 0:
            return jnp.concatenate([pad, x[:, : w - d]], axis=1)
        return jnp.concatenate([x[:, -d:], pad], axis=1)
    pad = jnp.full((abs(d), w), fill, x.dtype)
    if d > 0:
        return jnp.concatenate([pad, x[: h - d, :]], axis=0)
    return jnp.concatenate([x[-d:, :], pad], axis=0)


def _sum9(x, axis):
    acc = x
    for d in (-4, -3, -2, -1, 1, 2, 3, 4):
        acc = acc + _shift(x, d, axis, 0.0)
    return acc


def _max5(x, axis):
    ninf = -jnp.inf
    m3 = jnp.maximum(x, jnp.maximum(_shift(x, 1, axis, ninf),
                                    _shift(x, -1, axis, ninf)))
    return jnp.maximum(_shift(m3, 1, axis, ninf), _shift(m3, -1, axis, ninf))


def _body(vs_ref, dp_ref, *refs):
    b = vs_ref.shape[0]
    out_refs = refs[:b]
    mx_refs = refs[b:]
    h, w = mx_refs[0].shape
    nb = h // 8

    ciota = lax.broadcasted_iota(jnp.int32, (1, w), 1)
    xxc = lax.broadcasted_iota(jnp.int32, (1, _CSLAB), 1).astype(jnp.float32)
    fiota = (lax.broadcasted_iota(jnp.int32, (8, nb), 0) * nb
             + lax.broadcasted_iota(jnp.int32, (8, nb), 1))
    yyf = lax.broadcasted_iota(jnp.int32, (_SLAB, 1), 0).astype(jnp.float32)

    rm0 = []
    for bi in range(b):
        p = vs_ref[bi]
        occ = (p > 0.2).astype(jnp.float32)
        so = _sum9(_sum9(occ, 1), 0)
        sm = _sum9(_sum9(p, 1), 0)
        prod = so * sm * (1.0 / (81.0 * 81.0))
        mn = jnp.min(prod)
        mxv = jnp.max(prod)
        compact = (prod - mn) / (mxv - mn + 1e-6)
        score = p * compact
        pooled = _max5(_max5(score, 1), 0)
        maxima = jnp.where(score == pooled, score, 0.0)
        mx_refs[bi][:, :] = maxima
        rm0.append(jnp.max(maxima.reshape(8, nb, w), axis=2))
        out_refs[bi][:, :] = jnp.full((h, w), -jnp.inf, jnp.float32)

    def splat(bi, lv, rfl, ic, cfl, base, cbase):
        base = pl.multiple_of(base, 8)
        cbase = pl.multiple_of(cbase, 128)
        ay = lv - (yyf - rfl) ** 2 * ic
        bx = (xxc - cfl) ** 2 * ic
        cur = out_refs[bi][pl.ds(base, _SLAB), pl.ds(cbase, _CSLAB)]
        out_refs[bi][pl.ds(base, _SLAB), pl.ds(cbase, _CSLAB)] = (
            jnp.maximum(cur, ay - bx))

    def step1(bi, st):
        rm, plv, prfl, pic, pcfl, pbase, pcbase = st
        m = jnp.max(rm)
        r = jnp.min(jnp.where(rm == m, fiota, h))
        row = mx_refs[bi][pl.ds(r, 1), :]
        c = jnp.min(jnp.where(row == m, ciota, w))
        newrow = jnp.where(ciota == c, 0.0, row)
        mx_refs[bi][pl.ds(r, 1), :] = newrow
        rm = jnp.where(fiota == r, jnp.max(newrow), rm)
        splat(bi, plv, prfl, pic, pcfl, pbase, pcbase)
        drow = dp_ref[bi, pl.ds(r, 1), :]
        z = jnp.sum(jnp.where(ciota == c, drow, 0.0))
        z = jnp.maximum(z, 0.001)
        radius = jnp.clip(_RADIUS_GAIN / z, _RADIUS_MIN, _RADIUS_MAX)
        sig2 = (0.6 * radius) ** 2
        invc = 1.0 / (2.0 * sig2 + 1e-6)
        base = pl.multiple_of((jnp.clip(r - _SLAB // 2, 0, h - _SLAB) // 8) * 8,
                              8)
        cbase = (jnp.clip(c - 56, 0, w - _CSLAB) // 128) * 128
        return (rm, jnp.log(m), (r - base).astype(jnp.float32), invc,
                (c - cbase).astype(jnp.float32), base, cbase)

    def step(i, sts):
        return tuple(step1(bi, sts[bi]) for bi in range(b))

    init = tuple((rm0[bi], -jnp.inf, 0.0, 0.0, 0.0, 0, 0) for bi in range(b))
    fin = lax.fori_loop(0, _TOPK, step, init, unroll=4)

    for bi in range(b):
        splat(bi, *fin[bi][1:])
        g = jnp.exp(out_refs[bi][:, :])
        gmn = jnp.min(g)
        gmx = jnp.max(g)
        out_refs[bi][:, :] = (g - gmn) / (gmx - gmn + 1e-6)


def kernel(voxel_score_map, depth):
    b, ch, h, w = voxel_score_map.shape
    vs = voxel_score_map.reshape(b, h, w)
    dp = depth.reshape(b, h, w)
    outs = pl.pallas_call(
        _body,
        out_shape=[jax.ShapeDtypeStruct((h, w), jnp.float32)
                   for _ in range(b)],
        scratch_shapes=[pltpu.VMEM((h, w), jnp.float32) for _ in range(b)],
    )(vs, dp)
    return jnp.stack(outs, axis=0).reshape(b, ch, h, w)
